# Initial kernel scaffold; baseline (speedup 1.0000x reference)
import jax
import jax.numpy as jnp
from jax.experimental import pallas as pl


def kernel(x1, x2, adj1, adj2, Wx_w, Wx_b, Wn_w, Wn_b, Wr_w, Wr_b, gamma, beta):
    raise NotImplementedError("write your pallas kernel here")


# speedup vs baseline: 1.5926x; 1.5926x over previous
import jax
import jax.numpy as jnp
from jax.experimental import pallas as pl

_F32 = jnp.float32


def _make_body(B, N, P, D, DOUT):
    BLK = N
    PB = P * N

    def body(x1_ref, x2_ref, wx_ref, wn_ref, wr_ref,
             bx_ref, bn_ref, br_ref, g_ref, bt_ref,
             o1_ref, o2_ref):
        att1 = []
        att2t = []
        for b in range(B):
            num = jnp.zeros((N, N), _F32)
            sq1 = jnp.zeros((N, 1), _F32)
            sq2 = jnp.zeros((N, 1), _F32)
            for p in range(P):
                r = b * PB + p * BLK
                a1 = x1_ref[r:r + BLK, :]
                a2 = x2_ref[r:r + BLK, :]
                num += jax.lax.dot_general(
                    a1, a2, (((1,), (1,)), ((), ())),
                    preferred_element_type=_F32)
                sq1 += jnp.sum(a1 * a1, axis=1, keepdims=True)
                sq2 += jnp.sum(a2 * a2, axis=1, keepdims=True)
            n1 = jnp.maximum(jnp.sqrt(sq1), 1e-6)
            n2 = jnp.maximum(jnp.sqrt(sq2), 1e-6)
            sim = num / (n1 * n2.T)
            m1 = jnp.max(sim, axis=1, keepdims=True)
            e1 = jnp.exp(sim - m1)
            att1.append(e1 / jnp.sum(e1, axis=1, keepdims=True))
            m2 = jnp.max(sim, axis=0, keepdims=True)
            e2 = jnp.exp(sim - m2)
            att2t.append(e2 / jnp.sum(e2, axis=0, keepdims=True))

        wx = wx_ref[:]
        wn = wn_ref[:]
        wr = wr_ref[:]
        bx = bx_ref[:]
        bn = bn_ref[:]
        br = br_ref[:]
        X1 = x1_ref[:]
        X2 = x2_ref[:]

        o1_ref[:, 0:DOUT] = jnp.dot(X1, wx, preferred_element_type=_F32) + bx
        o2_ref[:, 0:DOUT] = jnp.dot(X2, wx, preferred_element_type=_F32) + bx

        inv = 1.0 / (N - 1)
        for X, oref in ((X1, o1_ref), (X2, o2_ref)):
            Z = jnp.dot(X, wn, preferred_element_type=_F32)
            Z3 = Z.reshape(B * P, BLK, DOUT)
            s = jnp.sum(Z3, axis=1, keepdims=True)
            XN = ((s - Z3) * inv).reshape(B * PB, DOUT)
            oref[:, DOUT:2 * DOUT] = XN + bn

        Z1r = jnp.dot(X1, wr, preferred_element_type=_F32)
        Z2r = jnp.dot(X2, wr, preferred_element_type=_F32)
        for b in range(B):
            a1 = att1[b]
            a2t = att2t[b]
            for p in range(P):
                r = b * PB + p * BLK
                c1 = jnp.dot(a1, Z2r[r:r + BLK, :], preferred_element_type=_F32)
                o1_ref[r:r + BLK, 2 * DOUT:3 * DOUT] = Z1r[r:r + BLK, :] - c1 + br
                c2 = jax.lax.dot_general(
                    a2t, Z1r[r:r + BLK, :], (((0,), (0,)), ((), ())),
                    preferred_element_type=_F32)
                o2_ref[r:r + BLK, 2 * DOUT:3 * DOUT] = Z2r[r:r + BLK, :] - c2 + br

        g = g_ref[:]
        bt = bt_ref[:]
        for oref in (o1_ref, o2_ref):
            h = oref[:]
            nrm = jnp.sqrt(jnp.sum(h * h, axis=1, keepdims=True))
            h = h / jnp.maximum(nrm, 1e-12)
            h = jnp.maximum(h, 0.0)
            mean = jnp.mean(h, axis=0, keepdims=True)
            var = jnp.mean((h - mean) ** 2, axis=0, keepdims=True)
            oref[:] = g * (h - mean) * jax.lax.rsqrt(var + 1e-5) + bt

    return body


def kernel(x1, x2, adj1, adj2, Wx_w, Wx_b, Wn_w, Wn_b, Wr_w, Wr_b, gamma, beta):
    B, N, P, D = x1.shape
    DOUT = Wx_w.shape[0]
    C = 3 * DOUT
    M = B * N * P

    x1p = x1.transpose(0, 2, 1, 3).reshape(M, D)
    x2p = x2.transpose(0, 2, 1, 3).reshape(M, D)

    out1, out2 = pl.pallas_call(
        _make_body(B, N, P, D, DOUT),
        out_shape=(
            jax.ShapeDtypeStruct((M, C), jnp.float32),
            jax.ShapeDtypeStruct((M, C), jnp.float32),
        ),
    )(x1p, x2p,
      Wx_w.T, Wn_w.T, Wr_w.T,
      Wx_b.reshape(1, DOUT), Wn_b.reshape(1, DOUT), Wr_b.reshape(1, DOUT),
      gamma.reshape(1, C), beta.reshape(1, C))

    o1 = out1.reshape(B, P, N, C).transpose(0, 2, 1, 3)
    o2 = out2.reshape(B, P, N, C).transpose(0, 2, 1, 3)
    return (o1, o2)
